# baseline (speedup 1.0000x reference)
import functools

import jax
import jax.numpy as jnp
from jax import lax
from jax.experimental import pallas as pl
from jax.experimental.pallas import tpu as pltpu
from jax.experimental.pallas import tpu_sc as plsc

V = 1000001
D = 64
T = 20
B = 16384
CPS = D // 2
BPT = B // 16
GPT = BPT // 128

_mesh = plsc.VectorSubcoreMesh(core_axis_name="c", subcore_axis_name="s")


@functools.partial(
    pl.kernel,
    mesh=_mesh,
    out_type=jax.ShapeDtypeStruct((T, D, B), jnp.float32),
    compiler_params=pltpu.CompilerParams(use_tc_tiling_on_sc=False),
    scratch_types=[
        pltpu.VMEM_SHARED((2, V), jnp.float32),
        pltpu.VMEM((2, GPT, 128), jnp.int32),
        pltpu.VMEM((2, BPT), jnp.float32),
        pltpu.SemaphoreType.DMA,
        pltpu.SemaphoreType.DMA,
        pltpu.SemaphoreType.DMA,
        pltpu.SemaphoreType.DMA,
    ],
)
def _emb_gather(wt, idx3, o, sp, idx_v, out_v, ssem, isem, gsem, osem):
    sc = lax.axis_index("c")
    w = lax.axis_index("s")
    b0 = w * BPT
    c_base = sc * CPS

    def idx_dma(i, slot):
        t = i % T
        return pltpu.make_async_copy(
            idx3.at[t, pl.ds(w * GPT, GPT), :], idx_v.at[slot], isem
        )

    def out_dma(i, slot):
        ci = i // T
        t = i % T
        return pltpu.make_async_copy(
            out_v.at[slot], o.at[t, c_base + ci, pl.ds(b0, BPT)], osem
        )

    def stage_dma(ci, slot):
        return pltpu.make_async_copy(wt.at[c_base + ci], sp.at[slot], ssem)

    @pl.when(w == 0)
    def _():
        stage_dma(0, 0).start()

    idx_dma(0, 0).start()

    def body(i, carry):
        ci = i // T
        t = i % T
        sbuf = ci % 2
        slot = i % 2

        @pl.when(t == 0)
        def _():
            @pl.when(w == 0)
            def _():
                stage_dma(ci, sbuf).wait()

            plsc.subcore_barrier()

            @pl.when((w == 0) & (ci + 1 < CPS))
            def _():
                stage_dma(ci + 1, 1 - sbuf).start()

        @pl.when(i + 1 < CPS * T)
        def _():
            idx_dma(i + 1, 1 - slot).start()

        idx_dma(i, slot).wait()

        @pl.when(i >= 2)
        def _():
            out_dma(i - 2, slot).wait()

        for j in range(GPT):
            pltpu.async_copy(
                sp.at[sbuf].at[idx_v.at[slot, j]],
                out_v.at[slot, pl.ds(j * 128, 128)],
                gsem,
            )
        for j in range(GPT):
            pltpu.make_async_copy(
                sp.at[sbuf].at[idx_v.at[slot, j]],
                out_v.at[slot, pl.ds(j * 128, 128)],
                gsem,
            ).wait()

        out_dma(i, slot).start()
        return carry

    n = CPS * T
    lax.fori_loop(0, n, body, 0)
    out_dma(n - 2, n % 2).wait()
    out_dma(n - 1, (n - 1) % 2).wait()


def kernel(input, weight):
    idx3 = input.T.reshape(T, B // 128, 128)
    o = _emb_gather(weight.T, idx3)
    return jnp.transpose(o, (2, 0, 1))


# speedup vs baseline: 1.8944x; 1.8944x over previous
import functools

import jax
import jax.numpy as jnp
from jax import lax
from jax.experimental import pallas as pl
from jax.experimental.pallas import tpu as pltpu
from jax.experimental.pallas import tpu_sc as plsc

V = 1000001
D = 64
T = 20
B = 16384
CPS = D // 2
BPT = B // 16
GPT = BPT // 128

VBLK = 8192
NVB = -(-V // VBLK)
CSTRIDE = NVB * VBLK
SLEN = 1000064

_mesh = plsc.VectorSubcoreMesh(core_axis_name="c", subcore_axis_name="s")


@functools.partial(
    pl.pallas_call,
    grid=(NVB, D),
    in_specs=[pl.BlockSpec((D, VBLK), lambda v, c: (0, v))],
    out_specs=pl.BlockSpec((VBLK,), lambda v, c: (c * NVB + v,)),
    out_shape=jax.ShapeDtypeStruct((D * CSTRIDE,), jnp.float32),
)
def _detile(wt_ref, o_ref):
    o_ref[...] = wt_ref[pl.program_id(1), :]


@functools.partial(
    pl.kernel,
    mesh=_mesh,
    out_type=jax.ShapeDtypeStruct((T, D, B), jnp.float32),
    compiler_params=pltpu.CompilerParams(use_tc_tiling_on_sc=False),
    scratch_types=[
        pltpu.VMEM_SHARED((2, SLEN), jnp.float32),
        pltpu.VMEM((2, GPT, 128), jnp.int32),
        pltpu.VMEM((2, BPT), jnp.float32),
        pltpu.SemaphoreType.DMA,
        pltpu.SemaphoreType.DMA,
        pltpu.SemaphoreType.DMA,
        pltpu.SemaphoreType.DMA,
    ],
)
def _emb_gather(wt_flat, idx3, o, sp, idx_v, out_v, ssem, isem, gsem, osem):
    sc = lax.axis_index("c")
    w = lax.axis_index("s")
    b0 = w * BPT
    c_base = sc * CPS

    def idx_dma(i, slot):
        t = i % T
        return pltpu.make_async_copy(
            idx3.at[t, pl.ds(w * GPT, GPT), :], idx_v.at[slot], isem
        )

    def out_dma(i, slot):
        ci = i // T
        t = i % T
        return pltpu.make_async_copy(
            out_v.at[slot], o.at[t, c_base + ci, pl.ds(b0, BPT)], osem
        )

    def stage_dma(ci, slot):
        return pltpu.make_async_copy(
            wt_flat.at[pl.ds((c_base + ci) * CSTRIDE, SLEN)], sp.at[slot], ssem
        )

    @pl.when(w == 0)
    def _():
        stage_dma(0, 0).start()

    idx_dma(0, 0).start()

    def body(i, carry):
        ci = i // T
        t = i % T
        sbuf = ci % 2
        slot = i % 2

        @pl.when(t == 0)
        def _():
            @pl.when(w == 0)
            def _():
                stage_dma(ci, sbuf).wait()

            plsc.subcore_barrier()

            @pl.when((w == 0) & (ci + 1 < CPS))
            def _():
                stage_dma(ci + 1, 1 - sbuf).start()

        @pl.when(i + 1 < CPS * T)
        def _():
            idx_dma(i + 1, 1 - slot).start()

        idx_dma(i, slot).wait()

        @pl.when(i >= 2)
        def _():
            out_dma(i - 2, slot).wait()

        for j in range(GPT):
            pltpu.async_copy(
                sp.at[sbuf].at[idx_v.at[slot, j]],
                out_v.at[slot, pl.ds(j * 128, 128)],
                gsem,
            )
        for j in range(GPT):
            pltpu.make_async_copy(
                sp.at[sbuf].at[idx_v.at[slot, j]],
                out_v.at[slot, pl.ds(j * 128, 128)],
                gsem,
            ).wait()

        out_dma(i, slot).start()
        return carry

    n = CPS * T
    lax.fori_loop(0, n, body, 0)
    out_dma(n - 2, n % 2).wait()
    out_dma(n - 1, (n - 1) % 2).wait()


def kernel(input, weight):
    wt_flat = _detile(weight.T)
    idx3 = input.T.reshape(T, B // 128, 128)
    o = _emb_gather(wt_flat, idx3)
    return jnp.transpose(o, (2, 0, 1))


# speedup vs baseline: 5.2505x; 2.7716x over previous
import functools

import jax
import jax.numpy as jnp
from jax import lax
from jax.experimental import pallas as pl
from jax.experimental.pallas import tpu as pltpu
from jax.experimental.pallas import tpu_sc as plsc

V = 1000001
D = 64
T = 20
B = 16384
CPS = D // 2
BPT = B // 16
GPT = BPT // 128

VBLK = 8192
NVB = -(-V // VBLK)
CSTRIDE = NVB * VBLK
SLEN = 1000064

_mesh = plsc.VectorSubcoreMesh(core_axis_name="c", subcore_axis_name="s")


@functools.partial(
    pl.pallas_call,
    grid=(D // 8, NVB),
    in_specs=[pl.BlockSpec((8, VBLK), lambda cb, v: (cb, v))],
    out_specs=[
        pl.BlockSpec((VBLK,), lambda cb, v: (cb * NVB + v,)) for _ in range(8)
    ],
    out_shape=[
        jax.ShapeDtypeStruct((D // 8 * CSTRIDE,), jnp.float32) for _ in range(8)
    ],
)
def _detile(wt_ref, *o_refs):
    for k in range(8):
        o_refs[k][...] = wt_ref[k, :]


@functools.partial(
    pl.kernel,
    mesh=_mesh,
    out_type=jax.ShapeDtypeStruct((T, D, B), jnp.float32),
    compiler_params=pltpu.CompilerParams(use_tc_tiling_on_sc=False),
    scratch_types=[
        pltpu.VMEM_SHARED((2, SLEN), jnp.float32),
        pltpu.VMEM((2, GPT, 128), jnp.int32),
        pltpu.VMEM((2, BPT), jnp.float32),
        pltpu.SemaphoreType.DMA,
        pltpu.SemaphoreType.DMA,
        pltpu.SemaphoreType.DMA,
        pltpu.SemaphoreType.DMA,
    ],
)
def _emb_gather(w0, w1, w2, w3, w4, w5, w6, w7, idx3, o,
                sp, idx_v, out_v, ssem, isem, gsem, osem):
    wts = (w0, w1, w2, w3, w4, w5, w6, w7)
    sc = lax.axis_index("c")
    w = lax.axis_index("s")
    b0 = w * BPT
    c_base = sc * CPS

    def idx_dma(i, slot):
        t = i % T
        return pltpu.make_async_copy(
            idx3.at[t, pl.ds(w * GPT, GPT), :], idx_v.at[slot], isem
        )

    def out_dma(i, slot):
        ci = i // T
        t = i % T
        return pltpu.make_async_copy(
            out_v.at[slot], o.at[t, c_base + ci, pl.ds(b0, BPT)], osem
        )

    def stage_dma_k(ci, slot, k):
        off = (sc * (CPS // 8) + ci // 8) * CSTRIDE
        return pltpu.make_async_copy(
            wts[k].at[pl.ds(off, SLEN)], sp.at[slot], ssem
        )

    def stage_start(ci, slot):
        for k in range(8):
            @pl.when(ci % 8 == k)
            def _():
                stage_dma_k(ci, slot, k).start()

    def stage_wait(ci, slot):
        for k in range(8):
            @pl.when(ci % 8 == k)
            def _():
                stage_dma_k(ci, slot, k).wait()

    @pl.when(w == 0)
    def _():
        stage_dma_k(0, 0, 0).start()

    idx_dma(0, 0).start()

    def body(i, carry):
        ci = i // T
        t = i % T
        sbuf = ci % 2
        slot = i % 2

        @pl.when(t == 0)
        def _():
            @pl.when(w == 0)
            def _():
                stage_wait(ci, sbuf)

            plsc.subcore_barrier()

            @pl.when((w == 0) & (ci + 1 < CPS))
            def _():
                stage_start(ci + 1, 1 - sbuf)

        @pl.when(i + 1 < CPS * T)
        def _():
            idx_dma(i + 1, 1 - slot).start()

        idx_dma(i, slot).wait()

        @pl.when(i >= 2)
        def _():
            out_dma(i - 2, slot).wait()

        for j in range(GPT):
            pltpu.async_copy(
                sp.at[sbuf].at[idx_v.at[slot, j]],
                out_v.at[slot, pl.ds(j * 128, 128)],
                gsem,
            )
        for j in range(GPT):
            pltpu.make_async_copy(
                sp.at[sbuf].at[idx_v.at[slot, j]],
                out_v.at[slot, pl.ds(j * 128, 128)],
                gsem,
            ).wait()

        out_dma(i, slot).start()
        return carry

    n = CPS * T
    lax.fori_loop(0, n, body, 0)
    out_dma(n - 2, n % 2).wait()
    out_dma(n - 1, (n - 1) % 2).wait()


def kernel(input, weight):
    wts = _detile(weight.T)
    idx3 = input.T.reshape(T, B // 128, 128)
    o = _emb_gather(*wts, idx3)
    return jnp.transpose(o, (2, 0, 1))
